# Initial kernel scaffold; baseline (speedup 1.0000x reference)
import jax
import jax.numpy as jnp
from jax.experimental import pallas as pl


def kernel(input, grid, input_mask, padding_buf):
    raise NotImplementedError("write your pallas kernel here")


# speedup vs baseline: 1.5123x; 1.5123x over previous
import jax
import jax.numpy as jnp
from jax import lax
from jax.experimental import pallas as pl
from jax.experimental.pallas import tpu as pltpu
from jax.experimental.pallas import tpu_sc as plsc

H = 512
W = 512
C = 96
HW = H * W

NC = 2
NS = 16
NW = NC * NS
PPW = HW // NW
P = 128
NCHUNK = PPW // P
L = 16
CB = C // L


def _bcast_lane(v, j):
    idx = jnp.full((L,), j, dtype=jnp.int32)
    return lax.gather(
        v, idx[:, None],
        lax.GatherDimensionNumbers(
            offset_dims=(), collapsed_slice_dims=(0,), start_index_map=(0,)),
        slice_sizes=(1,),
        mode=lax.GatherScatterMode.PROMISE_IN_BOUNDS)


def _sc_grid_sample(inp_t, gx, gy, pad):
    mesh = plsc.VectorSubcoreMesh(core_axis_name="c", subcore_axis_name="s")

    def body(inp_hbm, gx_hbm, gy_hbm, pad_hbm, out_hbm,
             gxv, gyv, padv,
             i00v, i01v, i10v, i11v,
             w00v, w01v, w10v, w11v, ptv,
             r00, r01, r10, r11, outv, sem):
        wid = lax.axis_index("s") * NC + lax.axis_index("c")
        base = wid * PPW

        def chunk(ci, _):
            off = base + ci * P
            pltpu.sync_copy(gx_hbm.at[pl.ds(off, P)], gxv)
            pltpu.sync_copy(gy_hbm.at[pl.ds(off, P)], gyv)
            pltpu.sync_copy(pad_hbm.at[pl.ds(off, P)], padv)

            for g in range(P // L):
                s = g * L
                gx16 = gxv[pl.ds(s, L)]
                gy16 = gyv[pl.ds(s, L)]
                ix = ((gx16 + 1.0) * W - 1.0) / 2.0
                iy = ((gy16 + 1.0) * H - 1.0) / 2.0
                tx = ix.astype(jnp.int32)
                ty = iy.astype(jnp.int32)
                x0 = jnp.where(ix < tx.astype(jnp.float32), tx - 1, tx)
                y0 = jnp.where(iy < ty.astype(jnp.float32), ty - 1, ty)
                wx1 = ix - x0.astype(jnp.float32)
                wy1 = iy - y0.astype(jnp.float32)
                wx0 = 1.0 - wx1
                wy0 = 1.0 - wy1
                x1 = x0 + 1
                y1 = y0 + 1
                vx0 = (x0 >= 0) & (x0 <= W - 1)
                vx1 = (x1 >= 0) & (x1 <= W - 1)
                vy0 = (y0 >= 0) & (y0 <= H - 1)
                vy1 = (y1 >= 0) & (y1 <= H - 1)
                zero = jnp.zeros((L,), jnp.float32)
                w00 = jnp.where(vy0 & vx0, wy0 * wx0, zero)
                w01 = jnp.where(vy0 & vx1, wy0 * wx1, zero)
                w10 = jnp.where(vy1 & vx0, wy1 * wx0, zero)
                w11 = jnp.where(vy1 & vx1, wy1 * wx1, zero)
                m = w00 + w01 + w10 + w11
                x0c = jnp.clip(x0, 0, W - 1)
                x1c = jnp.clip(x1, 0, W - 1)
                yb0 = jnp.clip(y0, 0, H - 1) * W
                yb1 = jnp.clip(y1, 0, H - 1) * W
                i00v[pl.ds(s, L)] = yb0 + x0c
                i01v[pl.ds(s, L)] = yb0 + x1c
                i10v[pl.ds(s, L)] = yb1 + x0c
                i11v[pl.ds(s, L)] = yb1 + x1c
                w00v[pl.ds(s, L)] = w00 * m
                w01v[pl.ds(s, L)] = w01 * m
                w10v[pl.ds(s, L)] = w10 * m
                w11v[pl.ds(s, L)] = w11 * m
                ptv[pl.ds(s, L)] = padv[pl.ds(s, L)] * (1.0 - m)

            c0 = pltpu.async_copy(inp_hbm.at[i00v], r00, sem)
            c1 = pltpu.async_copy(inp_hbm.at[i01v], r01, sem)
            c2 = pltpu.async_copy(inp_hbm.at[i10v], r10, sem)
            c3 = pltpu.async_copy(inp_hbm.at[i11v], r11, sem)
            c0.wait()
            c1.wait()
            c2.wait()
            c3.wait()

            def comb(g, _):
                s = g * L
                w00g = w00v[pl.ds(s, L)]
                w01g = w01v[pl.ds(s, L)]
                w10g = w10v[pl.ds(s, L)]
                w11g = w11v[pl.ds(s, L)]
                ptg = ptv[pl.ds(s, L)]
                for j in range(L):
                    p = s + j
                    b00 = _bcast_lane(w00g, j)
                    b01 = _bcast_lane(w01g, j)
                    b10 = _bcast_lane(w10g, j)
                    b11 = _bcast_lane(w11g, j)
                    bpt = _bcast_lane(ptg, j)
                    for cb in range(CB):
                        cs = cb * L
                        acc = b00 * r00[p, pl.ds(cs, L)] + bpt
                        acc = acc + b01 * r01[p, pl.ds(cs, L)]
                        acc = acc + b10 * r10[p, pl.ds(cs, L)]
                        acc = acc + b11 * r11[p, pl.ds(cs, L)]
                        outv[p, pl.ds(cs, L)] = acc
                return 0

            lax.fori_loop(0, P // L, comb, 0)
            pltpu.sync_copy(outv, out_hbm.at[pl.ds(off, P)])
            return 0

        lax.fori_loop(0, NCHUNK, chunk, 0)

    f = pl.kernel(
        body,
        out_type=jax.ShapeDtypeStruct((HW, C), jnp.float32),
        mesh=mesh,
        scratch_types=[
            pltpu.VMEM((P,), jnp.float32),
            pltpu.VMEM((P,), jnp.float32),
            pltpu.VMEM((P,), jnp.float32),
            pltpu.VMEM((P,), jnp.int32),
            pltpu.VMEM((P,), jnp.int32),
            pltpu.VMEM((P,), jnp.int32),
            pltpu.VMEM((P,), jnp.int32),
            pltpu.VMEM((P,), jnp.float32),
            pltpu.VMEM((P,), jnp.float32),
            pltpu.VMEM((P,), jnp.float32),
            pltpu.VMEM((P,), jnp.float32),
            pltpu.VMEM((P,), jnp.float32),
            pltpu.VMEM((P, C), jnp.float32),
            pltpu.VMEM((P, C), jnp.float32),
            pltpu.VMEM((P, C), jnp.float32),
            pltpu.VMEM((P, C), jnp.float32),
            pltpu.VMEM((P, C), jnp.float32),
            pltpu.SemaphoreType.DMA,
        ],
        compiler_params=pltpu.CompilerParams(use_tc_tiling_on_sc=False),
    )
    return f(inp_t, gx, gy, pad)


def kernel(input, grid, input_mask, padding_buf):
    inp_t = input[0].reshape(C, HW).T
    gx = grid[0, :, :, 0].reshape(HW)
    gy = grid[0, :, :, 1].reshape(HW)
    pad = padding_buf[0, 0].reshape(HW)
    out_t = _sc_grid_sample(inp_t, gx, gy, pad)
    return out_t.T.reshape(1, C, H, W)


# speedup vs baseline: 1.9285x; 1.2752x over previous
import jax
import jax.numpy as jnp
from jax import lax
from jax.experimental import pallas as pl
from jax.experimental.pallas import tpu as pltpu
from jax.experimental.pallas import tpu_sc as plsc

H = 512
W = 512
C = 96
HW = H * W

NC = 2
NS = 16
NW = NC * NS
PPW = HW // NW
P = 128
NCHUNK = PPW // P
L = 16
CB = C // L


def _bcast_lane(v, j):
    idx = jnp.full((L,), j, dtype=jnp.int32)
    return lax.gather(
        v, idx[:, None],
        lax.GatherDimensionNumbers(
            offset_dims=(), collapsed_slice_dims=(0,), start_index_map=(0,)),
        slice_sizes=(1,),
        mode=lax.GatherScatterMode.PROMISE_IN_BOUNDS)


def _sc_grid_sample(inp_t, gxyp):
    mesh = plsc.VectorSubcoreMesh(core_axis_name="c", subcore_axis_name="s")

    def body(inp_hbm, gxyp_hbm, out_hbm, stage, idxs, ws, rs, outv, gsems):
        wid = lax.axis_index("s") * NC + lax.axis_index("c")
        base = wid * PPW

        def fire(ci, s):
            pltpu.sync_copy(gxyp_hbm.at[pl.ds((base + ci * P) * 3, 3 * P)], stage)
            idxv = idxs[s]
            wv = ws[s]
            for g in range(P // L):
                o = g * L
                gx16 = stage[pl.ds(o, L)]
                gy16 = stage[pl.ds(P + o, L)]
                pad16 = stage[pl.ds(2 * P + o, L)]
                ix = ((gx16 + 1.0) * W - 1.0) / 2.0
                iy = ((gy16 + 1.0) * H - 1.0) / 2.0
                tx = ix.astype(jnp.int32)
                ty = iy.astype(jnp.int32)
                x0 = jnp.where(ix < tx.astype(jnp.float32), tx - 1, tx)
                y0 = jnp.where(iy < ty.astype(jnp.float32), ty - 1, ty)
                wx1 = ix - x0.astype(jnp.float32)
                wy1 = iy - y0.astype(jnp.float32)
                wx0 = 1.0 - wx1
                wy0 = 1.0 - wy1
                x1 = x0 + 1
                y1 = y0 + 1
                vx0 = (x0 >= 0) & (x0 <= W - 1)
                vx1 = (x1 >= 0) & (x1 <= W - 1)
                vy0 = (y0 >= 0) & (y0 <= H - 1)
                vy1 = (y1 >= 0) & (y1 <= H - 1)
                zero = jnp.zeros((L,), jnp.float32)
                w00 = jnp.where(vy0 & vx0, wy0 * wx0, zero)
                w01 = jnp.where(vy0 & vx1, wy0 * wx1, zero)
                w10 = jnp.where(vy1 & vx0, wy1 * wx0, zero)
                w11 = jnp.where(vy1 & vx1, wy1 * wx1, zero)
                m = w00 + w01 + w10 + w11
                x0c = jnp.clip(x0, 0, W - 1)
                x1c = jnp.clip(x1, 0, W - 1)
                yb0 = jnp.clip(y0, 0, H - 1) * W
                yb1 = jnp.clip(y1, 0, H - 1) * W
                idxv[0, pl.ds(o, L)] = yb0 + x0c
                idxv[1, pl.ds(o, L)] = yb0 + x1c
                idxv[2, pl.ds(o, L)] = yb1 + x0c
                idxv[3, pl.ds(o, L)] = yb1 + x1c
                wv[0, pl.ds(o, L)] = w00 * m
                wv[1, pl.ds(o, L)] = w01 * m
                wv[2, pl.ds(o, L)] = w10 * m
                wv[3, pl.ds(o, L)] = w11 * m
                wv[4, pl.ds(o, L)] = pad16 * (1.0 - m)
            for k in range(4):
                pltpu.async_copy(inp_hbm.at[idxv.at[k]], rs[s][k], gsems[s])

        def drain_combine(ci, s):
            off = base + ci * P
            for k in range(4):
                pltpu.make_async_copy(
                    inp_hbm.at[idxs[s].at[k]], rs[s][k], gsems[s]).wait()
            r0, r1, r2, r3 = rs[s]
            wv = ws[s]

            def comb(g, _):
                o = g * L
                w00g = wv[0, pl.ds(o, L)]
                w01g = wv[1, pl.ds(o, L)]
                w10g = wv[2, pl.ds(o, L)]
                w11g = wv[3, pl.ds(o, L)]
                ptg = wv[4, pl.ds(o, L)]
                for j in range(L):
                    p = o + j
                    b00 = _bcast_lane(w00g, j)
                    b01 = _bcast_lane(w01g, j)
                    b10 = _bcast_lane(w10g, j)
                    b11 = _bcast_lane(w11g, j)
                    bpt = _bcast_lane(ptg, j)
                    for cb in range(CB):
                        cs = cb * L
                        acc = b00 * r0[p, pl.ds(cs, L)] + bpt
                        acc = acc + b01 * r1[p, pl.ds(cs, L)]
                        acc = acc + b10 * r2[p, pl.ds(cs, L)]
                        acc = acc + b11 * r3[p, pl.ds(cs, L)]
                        outv[p, pl.ds(cs, L)] = acc
                return 0

            lax.fori_loop(0, P // L, comb, 0)
            pltpu.sync_copy(outv, out_hbm.at[pl.ds(off, P)])

        fire(0, 0)

        def body2(k2, _):
            ci = k2 * 2

            @pl.when(ci + 1 < NCHUNK)
            def _():
                fire(ci + 1, 1)

            drain_combine(ci, 0)

            @pl.when(ci + 2 < NCHUNK)
            def _():
                fire(ci + 2, 0)

            @pl.when(ci + 1 < NCHUNK)
            def _():
                drain_combine(ci + 1, 1)

            return 0

        lax.fori_loop(0, (NCHUNK + 1) // 2, body2, 0)

    f = pl.kernel(
        body,
        out_type=jax.ShapeDtypeStruct((HW, C), jnp.float32),
        mesh=mesh,
        scratch_types=[
            pltpu.VMEM((3 * P,), jnp.float32),
            [pltpu.VMEM((4, P), jnp.int32) for _ in range(2)],
            [pltpu.VMEM((5, P), jnp.float32) for _ in range(2)],
            [[pltpu.VMEM((P, C), jnp.float32) for _ in range(4)]
             for _ in range(2)],
            pltpu.VMEM((P, C), jnp.float32),
            [pltpu.SemaphoreType.DMA for _ in range(2)],
        ],
        compiler_params=pltpu.CompilerParams(use_tc_tiling_on_sc=False),
    )
    return f(inp_t, gxyp)


def kernel(input, grid, input_mask, padding_buf):
    inp_t = input[0].reshape(C, HW).T
    gx = grid[0, :, :, 0].reshape(-1, P)
    gy = grid[0, :, :, 1].reshape(-1, P)
    pad = padding_buf[0, 0].reshape(-1, P)
    gxyp = jnp.stack([gx, gy, pad], axis=1).reshape(-1)
    out_t = _sc_grid_sample(inp_t, gxyp)
    return out_t.T.reshape(1, C, H, W)
